# baseline (speedup 1.0000x reference)
import functools

import jax
import jax.numpy as jnp
from jax import lax
from jax.experimental import pallas as pl
from jax.experimental.pallas import tpu as pltpu
from jax.experimental.pallas import tpu_sc as plsc

N = 10000
E = 320000
FD = 128
NCLS = 64

NC = 2
NS = 16
NW = NC * NS
NP = 10240
EPT = 10240
E_PAD = NW * EPT
CH = 128
NCHUNK = EPT // CH
RPT = NP // NS

_MESH = plsc.VectorSubcoreMesh(core_axis_name="c", subcore_axis_name="s")


@functools.partial(
    pl.kernel,
    out_type=(
        jax.ShapeDtypeStruct((NW, NP), jnp.float32),
        jax.ShapeDtypeStruct((NW, NP), jnp.float32),
    ),
    mesh=_MESH,
    compiler_params=pltpu.CompilerParams(use_tc_tiling_on_sc=False,
                                         needs_layout_passes=False),
    scratch_types=[
        pltpu.VMEM((NCHUNK, CH), jnp.int32),
        pltpu.VMEM((NCHUNK, CH), jnp.int32),
        pltpu.VMEM((NP,), jnp.float32),
        pltpu.VMEM((NP,), jnp.float32),
    ],
)
def _deg_sc(src_hbm, dst_hbm, degs_hbm, degd_hbm,
            src_v, dst_v, degs_t, degd_t):
    c = lax.axis_index("c")
    s = lax.axis_index("s")
    wid = c * NS + s

    @pl.loop(0, NP // 16)
    def _(i):
        degs_t[pl.ds(i * 16, 16)] = jnp.zeros((16,), jnp.float32)
        degd_t[pl.ds(i * 16, 16)] = jnp.zeros((16,), jnp.float32)

    pltpu.sync_copy(src_hbm.at[wid], src_v)
    pltpu.sync_copy(dst_hbm.at[wid], dst_v)

    ones16 = jnp.full((16,), 1.0, jnp.float32)

    @pl.loop(0, NCHUNK)
    def _(i):
        @pl.loop(0, CH // 16)
        def _(j):
            si = src_v[i, pl.ds(j * 16, 16)]
            plsc.addupdate_scatter(degs_t, [si], ones16)
            di = dst_v[i, pl.ds(j * 16, 16)]
            plsc.addupdate_scatter(degd_t, [di], ones16)

    pltpu.sync_copy(degs_t, degs_hbm.at[wid])
    pltpu.sync_copy(degd_t, degd_hbm.at[wid])


_NB = 4
NCHUNK2 = E_PAD // NS // CH


def _make_agg_sc(D):
    Dh = D // 2

    @functools.partial(
        pl.kernel,
        out_type=jax.ShapeDtypeStruct((NC, NP, Dh), jnp.float32),
        mesh=_MESH,
        compiler_params=pltpu.CompilerParams(use_tc_tiling_on_sc=False),
        scratch_types=[
            pltpu.VMEM((NCHUNK2, CH), jnp.int32),
            pltpu.VMEM((NCHUNK2, CH), jnp.int32),
            pltpu.VMEM((_NB, CH, Dh), jnp.float32),
            pltpu.VMEM_SHARED((NP, Dh), jnp.float32),
            pltpu.SemaphoreType.DMA,
            pltpu.SemaphoreType.DMA,
        ],
    )
    def _agg_sc(h_hbm, src_hbm, dst_hbm, out_hbm,
                src_v, dst_v, rows_v, agg_sh, gsem, ssem):
        c = lax.axis_index("c")
        s = lax.axis_index("s")

        @pl.loop(0, CH)
        def _(i):
            @pl.loop(0, Dh // 16)
            def _(j):
                rows_v[0, i, pl.ds(j * 16, 16)] = jnp.zeros((16,), jnp.float32)

        pltpu.sync_copy(src_hbm.at[s], src_v)
        pltpu.sync_copy(dst_hbm.at[s], dst_v)

        @pl.loop(0, RPT // CH)
        def _(t):
            row = s * RPT + t * CH
            pltpu.sync_copy(rows_v.at[0], agg_sh.at[pl.ds(row, CH)])
        plsc.subcore_barrier()

        for b in range(_NB - 2):
            pltpu.async_copy(h_hbm.at[c].at[src_v.at[b]], rows_v.at[b], gsem)

        @pl.loop(0, NCHUNK2)
        def _(i):
            par = lax.rem(i, _NB)
            pltpu.make_async_copy(h_hbm.at[c].at[src_v.at[0]], rows_v.at[0],
                                  gsem).wait()

            @pl.when(i >= 2)
            def _():
                pltpu.make_async_copy(rows_v.at[0], agg_sh.at[dst_v.at[0]],
                                      ssem).wait()

            @pl.when(i + _NB - 2 < NCHUNK2)
            def _():
                j = i + _NB - 2
                pltpu.async_copy(h_hbm.at[c].at[src_v.at[j]],
                                 rows_v.at[lax.rem(j, _NB)], gsem)

            pltpu.async_copy(rows_v.at[par], agg_sh.at[dst_v.at[i]], ssem,
                             add=True)

        @pl.loop(0, 2)
        def _(i):
            pltpu.make_async_copy(rows_v.at[0], agg_sh.at[dst_v.at[0]],
                                  ssem).wait()
        plsc.subcore_barrier()

        @pl.loop(0, RPT // CH)
        def _(t):
            row = s * RPT + t * CH
            pltpu.sync_copy(agg_sh.at[pl.ds(row, CH)], out_hbm.at[c, pl.ds(row, CH)])

    return _agg_sc


_agg_sc_128 = _make_agg_sc(FD)
_agg_sc_64 = _make_agg_sc(NCLS)

BM = 1024


def _norm_from_parts(p_ref):
    deg = jnp.sum(p_ref[...], axis=1, keepdims=True)
    return lax.rsqrt(jnp.maximum(deg, 1.0))


def _tc1_body(f_ref, ds_ref, o_ref):
    h = f_ref[...] * _norm_from_parts(ds_ref)
    o_ref[0] = h[:, :FD // 2]
    o_ref[1] = h[:, FD // 2:]


def _tc1(feats, degs_p):
    return pl.pallas_call(
        _tc1_body,
        grid=(NP // BM,),
        in_specs=[
            pl.BlockSpec((BM, FD), lambda i: (i, 0)),
            pl.BlockSpec((BM, NW), lambda i: (i, 0)),
        ],
        out_specs=pl.BlockSpec((NC, BM, FD // 2), lambda i: (0, i, 0)),
        out_shape=jax.ShapeDtypeStruct((NC, NP, FD // 2), jnp.float32),
    )(feats, degs_p)


def _tc2_body(p_ref, ds_ref, dd_ref, w1_ref, b1_ref, w2_ref, o_ref):
    agg = jnp.concatenate([p_ref[0], p_ref[1]], axis=1)
    nsrc = _norm_from_parts(ds_ref)
    ndst = _norm_from_parts(dd_ref)
    x1 = jnp.dot(agg, w1_ref[...], preferred_element_type=jnp.float32)
    x1 = jnp.maximum(x1 * ndst + b1_ref[...], 0.0)
    y = jnp.dot(x1 * nsrc, w2_ref[...], preferred_element_type=jnp.float32)
    o_ref[0] = y[:, :NCLS // 2]
    o_ref[1] = y[:, NCLS // 2:]


def _tc2(p1, degs_p, degd_p, W1, b1, W2):
    return pl.pallas_call(
        _tc2_body,
        grid=(NP // BM,),
        in_specs=[
            pl.BlockSpec((NC, BM, FD // 2), lambda i: (0, i, 0)),
            pl.BlockSpec((BM, NW), lambda i: (i, 0)),
            pl.BlockSpec((BM, NW), lambda i: (i, 0)),
            pl.BlockSpec((FD, FD), lambda i: (0, 0)),
            pl.BlockSpec((1, FD), lambda i: (0, 0)),
            pl.BlockSpec((FD, NCLS), lambda i: (0, 0)),
        ],
        out_specs=pl.BlockSpec((NC, BM, NCLS // 2), lambda i: (0, i, 0)),
        out_shape=jax.ShapeDtypeStruct((NC, NP, NCLS // 2), jnp.float32),
    )(p1, degs_p, degd_p, W1, b1, W2)


def _tc3_body(p_ref, dd_ref, b2_ref, o_ref):
    agg = jnp.concatenate([p_ref[0], p_ref[1]], axis=1)
    o_ref[...] = agg * _norm_from_parts(dd_ref) + b2_ref[...]


def _tc3(p2, degd_p, b2):
    return pl.pallas_call(
        _tc3_body,
        grid=(NP // BM,),
        in_specs=[
            pl.BlockSpec((NC, BM, NCLS // 2), lambda i: (0, i, 0)),
            pl.BlockSpec((BM, NW), lambda i: (i, 0)),
            pl.BlockSpec((1, NCLS), lambda i: (0, 0)),
        ],
        out_specs=pl.BlockSpec((BM, NCLS), lambda i: (i, 0)),
        out_shape=jax.ShapeDtypeStruct((NP, NCLS), jnp.float32),
    )(p2, degd_p, b2)


def kernel(features, edge_index, W1, b1, W2, b2):
    src = edge_index[0].astype(jnp.int32)
    dst = edge_index[1].astype(jnp.int32)
    pad = jnp.full((E_PAD - E,), N, jnp.int32)
    src_flat = jnp.concatenate([src, pad])
    dst_flat = jnp.concatenate([dst, pad])
    src_d = src_flat.reshape(NW, NCHUNK, CH)
    dst_d = dst_flat.reshape(NW, NCHUNK, CH)
    src_a = src_flat.reshape(NS, NCHUNK2, CH)
    dst_a = dst_flat.reshape(NS, NCHUNK2, CH)
    feats_p = jnp.zeros((NP, FD), jnp.float32).at[:N].set(features)

    degs_w, degd_w = _deg_sc(src_d, dst_d)
    degs_p = degs_w.T
    degd_p = degd_w.T
    h1 = _tc1(feats_p, degs_p)
    p1 = _agg_sc_128(h1, src_a, dst_a)
    y2 = _tc2(p1, degs_p, degd_p, W1, b1.reshape(1, FD), W2)
    p2 = _agg_sc_64(y2, src_a, dst_a)
    out = _tc3(p2, degd_p, b2.reshape(1, NCLS))
    return out[:N]


# speedup vs baseline: 1.6472x; 1.6472x over previous
import functools

import jax
import jax.numpy as jnp
from jax import lax
from jax.experimental import pallas as pl
from jax.experimental.pallas import tpu as pltpu
from jax.experimental.pallas import tpu_sc as plsc

N = 10000
E = 320000
FD = 128
NCLS = 64

NC = 2
NS = 16
NW = NC * NS
NP = 10240
EPT = 10240
E_PAD = NW * EPT
CH = 128
NCHUNK = EPT // CH
RPT = NP // NS

_MESH = plsc.VectorSubcoreMesh(core_axis_name="c", subcore_axis_name="s")


@functools.partial(
    pl.kernel,
    out_type=(
        jax.ShapeDtypeStruct((NW, NP), jnp.float32),
        jax.ShapeDtypeStruct((NW, NP), jnp.float32),
    ),
    mesh=_MESH,
    compiler_params=pltpu.CompilerParams(use_tc_tiling_on_sc=False,
                                         needs_layout_passes=False),
    scratch_types=[
        pltpu.VMEM((NCHUNK, CH), jnp.int32),
        pltpu.VMEM((NCHUNK, CH), jnp.int32),
        pltpu.VMEM((NP,), jnp.float32),
        pltpu.VMEM((NP,), jnp.float32),
    ],
)
def _deg_sc(src_hbm, dst_hbm, degs_hbm, degd_hbm,
            src_v, dst_v, degs_t, degd_t):
    c = lax.axis_index("c")
    s = lax.axis_index("s")
    wid = c * NS + s

    @pl.loop(0, NP // 16)
    def _(i):
        degs_t[pl.ds(i * 16, 16)] = jnp.zeros((16,), jnp.float32)
        degd_t[pl.ds(i * 16, 16)] = jnp.zeros((16,), jnp.float32)

    pltpu.sync_copy(src_hbm.at[wid], src_v)
    pltpu.sync_copy(dst_hbm.at[wid], dst_v)

    ones16 = jnp.full((16,), 1.0, jnp.float32)

    @pl.loop(0, NCHUNK)
    def _(i):
        @pl.loop(0, CH // 16)
        def _(j):
            si = src_v[i, pl.ds(j * 16, 16)]
            plsc.addupdate_scatter(degs_t, [si], ones16)
            di = dst_v[i, pl.ds(j * 16, 16)]
            plsc.addupdate_scatter(degd_t, [di], ones16)

    pltpu.sync_copy(degs_t, degs_hbm.at[wid])
    pltpu.sync_copy(degd_t, degd_hbm.at[wid])


_NB = 4
NCHUNK2 = E_PAD // NS // CH
QN = 4
QCH = NCHUNK2 // QN


def _make_agg_sc(D):
    Dh = D // 2

    @functools.partial(
        pl.kernel,
        out_type=jax.ShapeDtypeStruct((NC, NP, Dh), jnp.float32),
        mesh=_MESH,
        compiler_params=pltpu.CompilerParams(use_tc_tiling_on_sc=False),
        scratch_types=[
            pltpu.VMEM((QCH, CH), jnp.int32),
            pltpu.VMEM((QCH, CH), jnp.int32),
            pltpu.VMEM((_NB, CH, Dh), jnp.float32),
            pltpu.VMEM_SHARED((NP, Dh), jnp.float32),
            pltpu.VMEM_SHARED((NP, Dh), jnp.float32),
            pltpu.SemaphoreType.DMA,
        ],
    )
    def _agg_sc(h_hbm, src_hbm, dst_hbm, out_hbm,
                src_v, dst_v, rows_v, h_sh, agg_sh, gsem):
        c = lax.axis_index("c")
        s = lax.axis_index("s")

        @pl.loop(0, CH)
        def _(i):
            @pl.loop(0, Dh // 16)
            def _(j):
                rows_v[0, i, pl.ds(j * 16, 16)] = jnp.zeros((16,), jnp.float32)

        @pl.loop(0, RPT // CH)
        def _(t):
            row = s * RPT + t * CH
            pltpu.sync_copy(rows_v.at[0], agg_sh.at[pl.ds(row, CH)])
            pltpu.sync_copy(h_hbm.at[c, pl.ds(row, CH)], h_sh.at[pl.ds(row, CH)])
        plsc.subcore_barrier()

        for q in range(QN):
            pltpu.sync_copy(src_hbm.at[s, q], src_v)
            pltpu.sync_copy(dst_hbm.at[s, q], dst_v)

            for b in range(_NB - 1):
                pltpu.async_copy(h_sh.at[src_v.at[b]], rows_v.at[b], gsem)

            @pl.loop(0, QCH)
            def _(i):
                par = lax.rem(i, _NB)
                pltpu.make_async_copy(h_sh.at[src_v.at[0]], rows_v.at[0],
                                      gsem).wait()

                @pl.when(i + _NB - 1 < QCH)
                def _():
                    j = i + _NB - 1
                    pltpu.async_copy(h_sh.at[src_v.at[j]],
                                     rows_v.at[lax.rem(j, _NB)], gsem)

                pltpu.sync_copy(rows_v.at[par], agg_sh.at[dst_v.at[i]],
                                add=True)

        plsc.subcore_barrier()

        @pl.loop(0, RPT // CH)
        def _(t):
            row = s * RPT + t * CH
            pltpu.sync_copy(agg_sh.at[pl.ds(row, CH)], out_hbm.at[c, pl.ds(row, CH)])

    return _agg_sc


_agg_sc_128 = _make_agg_sc(FD)
_agg_sc_64 = _make_agg_sc(NCLS)

BM = 1024


def _norm_from_parts(p_ref):
    deg = jnp.sum(p_ref[...], axis=1, keepdims=True)
    return lax.rsqrt(jnp.maximum(deg, 1.0))


def _tc1_body(f_ref, ds_ref, o_ref):
    h = f_ref[...] * _norm_from_parts(ds_ref)
    o_ref[0] = h[:, :FD // 2]
    o_ref[1] = h[:, FD // 2:]


def _tc1(feats, degs_p):
    return pl.pallas_call(
        _tc1_body,
        grid=(NP // BM,),
        in_specs=[
            pl.BlockSpec((BM, FD), lambda i: (i, 0)),
            pl.BlockSpec((BM, NW), lambda i: (i, 0)),
        ],
        out_specs=pl.BlockSpec((NC, BM, FD // 2), lambda i: (0, i, 0)),
        out_shape=jax.ShapeDtypeStruct((NC, NP, FD // 2), jnp.float32),
    )(feats, degs_p)


def _tc2_body(p_ref, ds_ref, dd_ref, w1_ref, b1_ref, w2_ref, o_ref):
    agg = jnp.concatenate([p_ref[0], p_ref[1]], axis=1)
    nsrc = _norm_from_parts(ds_ref)
    ndst = _norm_from_parts(dd_ref)
    x1 = jnp.dot(agg, w1_ref[...], preferred_element_type=jnp.float32)
    x1 = jnp.maximum(x1 * ndst + b1_ref[...], 0.0)
    y = jnp.dot(x1 * nsrc, w2_ref[...], preferred_element_type=jnp.float32)
    o_ref[0] = y[:, :NCLS // 2]
    o_ref[1] = y[:, NCLS // 2:]


def _tc2(p1, degs_p, degd_p, W1, b1, W2):
    return pl.pallas_call(
        _tc2_body,
        grid=(NP // BM,),
        in_specs=[
            pl.BlockSpec((NC, BM, FD // 2), lambda i: (0, i, 0)),
            pl.BlockSpec((BM, NW), lambda i: (i, 0)),
            pl.BlockSpec((BM, NW), lambda i: (i, 0)),
            pl.BlockSpec((FD, FD), lambda i: (0, 0)),
            pl.BlockSpec((1, FD), lambda i: (0, 0)),
            pl.BlockSpec((FD, NCLS), lambda i: (0, 0)),
        ],
        out_specs=pl.BlockSpec((NC, BM, NCLS // 2), lambda i: (0, i, 0)),
        out_shape=jax.ShapeDtypeStruct((NC, NP, NCLS // 2), jnp.float32),
    )(p1, degs_p, degd_p, W1, b1, W2)


def _tc3_body(p_ref, dd_ref, b2_ref, o_ref):
    agg = jnp.concatenate([p_ref[0], p_ref[1]], axis=1)
    o_ref[...] = agg * _norm_from_parts(dd_ref) + b2_ref[...]


def _tc3(p2, degd_p, b2):
    return pl.pallas_call(
        _tc3_body,
        grid=(NP // BM,),
        in_specs=[
            pl.BlockSpec((NC, BM, NCLS // 2), lambda i: (0, i, 0)),
            pl.BlockSpec((BM, NW), lambda i: (i, 0)),
            pl.BlockSpec((1, NCLS), lambda i: (0, 0)),
        ],
        out_specs=pl.BlockSpec((BM, NCLS), lambda i: (i, 0)),
        out_shape=jax.ShapeDtypeStruct((NP, NCLS), jnp.float32),
    )(p2, degd_p, b2)


def kernel(features, edge_index, W1, b1, W2, b2):
    src = edge_index[0].astype(jnp.int32)
    dst = edge_index[1].astype(jnp.int32)
    pad = jnp.full((E_PAD - E,), N, jnp.int32)
    src_flat = jnp.concatenate([src, pad])
    dst_flat = jnp.concatenate([dst, pad])
    src_d = src_flat.reshape(NW, NCHUNK, CH)
    dst_d = dst_flat.reshape(NW, NCHUNK, CH)
    src_a = src_flat.reshape(NS, QN, QCH, CH)
    dst_a = dst_flat.reshape(NS, QN, QCH, CH)
    feats_p = jnp.zeros((NP, FD), jnp.float32).at[:N].set(features)

    degs_w, degd_w = _deg_sc(src_d, dst_d)
    degs_p = degs_w.T
    degd_p = degd_w.T
    h1 = _tc1(feats_p, degs_p)
    p1 = _agg_sc_128(h1, src_a, dst_a)
    y2 = _tc2(p1, degs_p, degd_p, W1, b1.reshape(1, FD), W2)
    p2 = _agg_sc_64(y2, src_a, dst_a)
    out = _tc3(p2, degd_p, b2.reshape(1, NCLS))
    return out[:N]
